# Initial kernel scaffold; baseline (speedup 1.0000x reference)
import jax
import jax.numpy as jnp
from jax.experimental import pallas as pl


def kernel(x, adjs, W, b):
    raise NotImplementedError("write your pallas kernel here")


# speedup vs baseline: 2.4347x; 2.4347x over previous
import jax
import jax.numpy as jnp
from jax.experimental import pallas as pl

B, C_IN, N = 16, 128, 1024
C_OUT = 256
BPS = 4


def _gnn_body(x_ref, a_ref, w_ref, b_ref, y_ref):
    a0 = a_ref[0].astype(jnp.bfloat16)
    a1 = a_ref[1].astype(jnp.bfloat16)
    xb = x_ref[...].reshape(BPS * C_IN, N).astype(jnp.bfloat16)

    h1 = jnp.dot(xb, a0, preferred_element_type=jnp.float32).astype(jnp.bfloat16)
    h3 = jnp.dot(xb, a1, preferred_element_type=jnp.float32).astype(jnp.bfloat16)
    h2 = jnp.dot(h1, a0, preferred_element_type=jnp.float32).astype(jnp.bfloat16)
    h4 = jnp.dot(h3, a1, preferred_element_type=jnp.float32).astype(jnp.bfloat16)

    w16 = w_ref[...].astype(jnp.bfloat16)
    bias = b_ref[...]
    for i in range(BPS):
        s = slice(i * C_IN, (i + 1) * C_IN)
        xc = jnp.concatenate([xb[s], h1[s], h2[s], h3[s], h4[s]], axis=0)
        y_ref[i] = jnp.dot(w16, xc, preferred_element_type=jnp.float32) + bias


def kernel(x, adjs, W, b):
    b2d = b.reshape(C_OUT, 1)
    grid = (B // BPS,)
    return pl.pallas_call(
        _gnn_body,
        grid=grid,
        in_specs=[
            pl.BlockSpec((BPS, C_IN, N), lambda i: (i, 0, 0)),
            pl.BlockSpec((2, N, N), lambda i: (0, 0, 0)),
            pl.BlockSpec((C_OUT, 5 * C_IN), lambda i: (0, 0)),
            pl.BlockSpec((C_OUT, 1), lambda i: (0, 0)),
        ],
        out_specs=pl.BlockSpec((BPS, C_OUT, N), lambda i: (i, 0, 0)),
        out_shape=jax.ShapeDtypeStruct((B, C_OUT, N), jnp.float32),
    )(x, adjs, W, b2d)
